# baseline (speedup 1.0000x reference)
import functools

import jax
import jax.numpy as jnp
from jax import lax
from jax.experimental import pallas as pl
from jax.experimental.pallas import tpu as pltpu
from jax.experimental.pallas import tpu_sc as plsc

_B, _S = 4, 8192
_T = _B * _S
_D = 64
_H = 1024
_V = 409
_VS = (9, 16, 64, 64, 256)
_OFF = (0, 9, 25, 89, 153)
_NCOMP = 5

_NC, _NS = 2, 16
_NW = _NC * _NS
_TW = _T // _NW
_L = 16


def _sc_weighted(ids, emb_flat, scales_b):
    mesh = plsc.VectorSubcoreMesh(core_axis_name="c", subcore_axis_name="s")

    @functools.partial(
        pl.kernel,
        out_type=jax.ShapeDtypeStruct((_T * _D,), jnp.float32),
        mesh=mesh,
        scratch_types=[
            pltpu.VMEM((_V * _D,), jnp.float32),
            pltpu.VMEM((_NCOMP * _TW,), jnp.int32),
            pltpu.VMEM((_TW * _D,), jnp.float32),
            pltpu.VMEM((_NCOMP, _L), jnp.float32),
        ],
        compiler_params=pltpu.CompilerParams(
            use_tc_tiling_on_sc=False, needs_layout_passes=False),
    )
    def body(ids_hbm, emb_hbm, scales_hbm, w_hbm, table, idbuf, wbuf, scv):
        wid = lax.axis_index("s") * _NC + lax.axis_index("c")
        base = wid * _TW
        pltpu.sync_copy(emb_hbm, table)
        pltpu.sync_copy(scales_hbm, scv)
        for c in range(_NCOMP):
            pltpu.sync_copy(ids_hbm.at[pl.ds(c * _T + base, _TW)],
                            idbuf.at[pl.ds(c * _TW, _TW)])

        for c in range(_NCOMP):
            sv = scv[c, :]

            @plsc.parallel_loop(0, _VS[c], unroll=4)
            def sbody(r, c=c, sv=sv):
                o = (_OFF[c] + r) * _D
                for f in range(_D // _L):
                    fo = o + f * _L
                    table[pl.ds(fo, _L)] = table[pl.ds(fo, _L)] * sv

        for c in range(_NCOMP):
            lo = c * _TW

            @plsc.parallel_loop(0, _TW // _L, unroll=4)
            def tbody(j, lo=lo, c=c):
                o = lo + j * _L
                v = idbuf[pl.ds(o, _L)]
                v = (jnp.minimum(jnp.maximum(v, 0), _VS[c] - 1) + _OFF[c]) * _D
                idbuf[pl.ds(o, _L)] = v

        lanes = lax.iota(jnp.int32, _L)
        t64 = lanes * _D

        def gbody(g, _):
            tokm = g * (_L * _D) + t64
            rows = [idbuf[pl.ds(c * _TW + g * _L, _L)] for c in range(_NCOMP)]

            @plsc.parallel_loop(0, _D, unroll=16)
            def fbody(j):
                perm = (j + lanes) & (_D - 1)
                acc = plsc.load_gather(table, [rows[0] + perm])
                for c in range(1, _NCOMP):
                    acc = acc + plsc.load_gather(table, [rows[c] + perm])
                plsc.store_scatter(wbuf, [tokm + perm], acc)

            return 0

        lax.fori_loop(0, _TW // _L, gbody, 0)
        pltpu.sync_copy(wbuf, w_hbm.at[pl.ds(base * _D, _TW * _D)])

    return body(ids, emb_flat, scales_b)


def _tc_up_proj(w, up_t):
    tb = 4096

    def mm(w_ref, u_ref, o_ref):
        o_ref[...] = jnp.dot(w_ref[...], u_ref[...],
                             preferred_element_type=jnp.float32)

    return pl.pallas_call(
        mm,
        grid=(_T // tb,),
        in_specs=[
            pl.BlockSpec((tb, _D), lambda i: (i, 0)),
            pl.BlockSpec((_D, _H), lambda i: (0, 0)),
        ],
        out_specs=pl.BlockSpec((tb, _H), lambda i: (i, 0)),
        out_shape=jax.ShapeDtypeStruct((_T, _H), jnp.float32),
        compiler_params=pltpu.CompilerParams(
            dimension_semantics=("arbitrary",)),
    )(w, up_t)


def kernel(structure_ids, dep_levels, ast_depth_ids, sibling_index_ids,
           node_type_ids, emb_weight, up_proj_weight, component_scales):
    ids = jnp.concatenate(
        [a.reshape(-1) for a in (structure_ids, dep_levels, ast_depth_ids,
                                 sibling_index_ids, node_type_ids)], axis=0)
    scales_b = jnp.broadcast_to(
        component_scales.reshape(_NCOMP, 1).astype(jnp.float32), (_NCOMP, _L))
    w = _sc_weighted(ids, emb_weight.reshape(-1), scales_b)
    out = _tc_up_proj(w.reshape(_T, _D), up_proj_weight.T)
    return out.reshape(_B, _S, _H)


# speedup vs baseline: 1.0749x; 1.0749x over previous
import functools

import jax
import jax.numpy as jnp
from jax import lax
from jax.experimental import pallas as pl
from jax.experimental.pallas import tpu as pltpu
from jax.experimental.pallas import tpu_sc as plsc

_B, _S = 4, 8192
_T = _B * _S
_D = 64
_H = 1024
_V = 409
_VS = (9, 16, 64, 64, 256)
_OFF = (0, 9, 25, 89, 153)
_NCOMP = 5

_NC, _NS = 2, 16
_NW = _NC * _NS
_TW = _T // _NW
_L = 16


def _sc_weighted(ids, emb_flat, scales_b):
    mesh = plsc.VectorSubcoreMesh(core_axis_name="c", subcore_axis_name="s")

    @functools.partial(
        pl.kernel,
        out_type=jax.ShapeDtypeStruct((_T * _D,), jnp.float32),
        mesh=mesh,
        scratch_types=[
            pltpu.VMEM((_V * _D,), jnp.float32),
            pltpu.VMEM((_NCOMP * _TW,), jnp.int32),
            pltpu.VMEM((_TW * _D,), jnp.float32),
            pltpu.VMEM((_NCOMP, _L), jnp.float32),
        ],
        compiler_params=pltpu.CompilerParams(
            use_tc_tiling_on_sc=False, needs_layout_passes=False),
    )
    def body(ids_hbm, emb_hbm, scales_hbm, w_hbm, table, idbuf, wbuf, scv):
        wid = lax.axis_index("s") * _NC + lax.axis_index("c")
        base = wid * _TW
        pltpu.sync_copy(emb_hbm, table)
        pltpu.sync_copy(scales_hbm, scv)
        for c in range(_NCOMP):
            pltpu.sync_copy(ids_hbm.at[pl.ds(c * _T + base, _TW)],
                            idbuf.at[pl.ds(c * _TW, _TW)])

        for c in range(_NCOMP):
            sv = scv[c, :]

            @plsc.parallel_loop(0, _VS[c], unroll=4)
            def sbody(r, c=c, sv=sv):
                o = (_OFF[c] + r) * _D
                for f in range(_D // _L):
                    fo = o + f * _L
                    table[pl.ds(fo, _L)] = table[pl.ds(fo, _L)] * sv

        for c in range(_NCOMP):
            lo = c * _TW

            @plsc.parallel_loop(0, _TW // _L, unroll=4)
            def tbody(j, lo=lo, c=c):
                o = lo + j * _L
                v = idbuf[pl.ds(o, _L)]
                v = (jnp.minimum(jnp.maximum(v, 0), _VS[c] - 1) + _OFF[c]) * _D
                idbuf[pl.ds(o, _L)] = v

        lanes = lax.iota(jnp.int32, _L)
        t64 = lanes * _D

        def gbody(g, _):
            tokm = g * (_L * _D) + t64
            rows = [idbuf[pl.ds(c * _TW + g * _L, _L)] for c in range(_NCOMP)]

            @plsc.parallel_loop(0, _D, unroll=8)
            def fbody(j):
                perm = (j + lanes) & (_D - 1)
                acc = plsc.load_gather(table, [rows[0] + perm])
                for c in range(1, _NCOMP):
                    acc = acc + plsc.load_gather(table, [rows[c] + perm])
                plsc.store_scatter(wbuf, [tokm + perm], acc)

            return 0

        lax.fori_loop(0, _TW // _L, gbody, 0)
        pltpu.sync_copy(wbuf, w_hbm.at[pl.ds(base * _D, _TW * _D)])

    return body(ids, emb_flat, scales_b)


def _tc_up_proj(w, up_t):
    tb = 4096

    def mm(w_ref, u_ref, o_ref):
        o_ref[...] = jnp.dot(w_ref[...], u_ref[...],
                             preferred_element_type=jnp.float32)

    return pl.pallas_call(
        mm,
        grid=(_T // tb,),
        in_specs=[
            pl.BlockSpec((tb, _D), lambda i: (i, 0)),
            pl.BlockSpec((_D, _H), lambda i: (0, 0)),
        ],
        out_specs=pl.BlockSpec((tb, _H), lambda i: (i, 0)),
        out_shape=jax.ShapeDtypeStruct((_T, _H), jnp.float32),
        compiler_params=pltpu.CompilerParams(
            dimension_semantics=("arbitrary",)),
    )(w, up_t)


def kernel(structure_ids, dep_levels, ast_depth_ids, sibling_index_ids,
           node_type_ids, emb_weight, up_proj_weight, component_scales):
    ids = jnp.concatenate(
        [a.reshape(-1) for a in (structure_ids, dep_levels, ast_depth_ids,
                                 sibling_index_ids, node_type_ids)], axis=0)
    scales_b = jnp.broadcast_to(
        component_scales.reshape(_NCOMP, 1).astype(jnp.float32), (_NCOMP, _L))
    w = _sc_weighted(ids, emb_weight.reshape(-1), scales_b)
    out = _tc_up_proj(w.reshape(_T, _D), up_proj_weight.T)
    return out.reshape(_B, _S, _H)
